# Initial kernel scaffold; baseline (speedup 1.0000x reference)
import jax
import jax.numpy as jnp
from jax.experimental import pallas as pl


def kernel(x, pos, descriptor, params, batch):
    raise NotImplementedError("write your pallas kernel here")


# speedup vs baseline: 4.3615x; 4.3615x over previous
import functools
import math

import jax
import jax.numpy as jnp
from jax import lax
from jax.experimental import pallas as pl
from jax.experimental.pallas import tpu as pltpu
from jax.experimental.pallas import tpu_sc as plsc

N_PAD = 10240
R_KNN = 256
R_XC = 512
KNN_K = 3
INV = 1.0 / math.sqrt(1.0 + 1e-5)

LAYERS = [
    (57, 114, 64, 128),
    (114, 228, 128, 240),
    (228, 300, 240, 304),
    (300, 300, 304, 304),
]


def _mm(a, b, precision=lax.Precision.HIGHEST):
    return lax.dot_general(a, b, (((1,), (0,)), ((), ())),
                           precision=precision,
                           preferred_element_type=jnp.float32)


def _mmb(a, b):
    return lax.dot_general(a.astype(jnp.bfloat16), b.astype(jnp.bfloat16),
                           (((1,), (0,)), ((), ())),
                           preferred_element_type=jnp.float32)


def _bf(v):
    return v.astype(jnp.bfloat16).astype(jnp.float32)


def _elu(x):
    return jnp.where(x > 0, x, jnp.exp(jnp.minimum(x, 0.0)) - 1.0)


def _knn_body(posb_ref, post_ref, sqr_ref, sqc_ref, batb_ref, batr_ref,
              out_ref):
    posb = posb_ref[...]
    post = post_ref[...]
    bb = batb_ref[...]
    br = batr_ref[...]
    d2 = sqr_ref[...] + sqc_ref[...] - 2.0 * _mm(
        posb, post, precision=lax.Precision.DEFAULT)
    d2 = jnp.where(bb == br, d2, jnp.inf)
    iota = lax.broadcasted_iota(jnp.int32, d2.shape, 1)
    cols = []
    for _ in range(KNN_K):
        m = jnp.min(d2, axis=1, keepdims=True)
        idx = jnp.min(jnp.where(d2 == m, iota, jnp.int32(2 ** 30)), axis=1)
        cols.append(idx[:, None])
        d2 = jnp.where(iota == idx[:, None], jnp.inf, d2)
    out_ref[...] = jnp.concatenate(cols, axis=1)


def _knn(pos16, post, sqcol, sqrow, batcol, batrow):
    return pl.pallas_call(
        _knn_body,
        grid=(N_PAD // R_KNN,),
        in_specs=[
            pl.BlockSpec((R_KNN, 16), lambda i: (i, 0)),
            pl.BlockSpec((16, N_PAD), lambda i: (0, 0)),
            pl.BlockSpec((R_KNN, 1), lambda i: (i, 0)),
            pl.BlockSpec((1, N_PAD), lambda i: (0, 0)),
            pl.BlockSpec((R_KNN, 1), lambda i: (i, 0)),
            pl.BlockSpec((1, N_PAD), lambda i: (0, 0)),
        ],
        out_specs=pl.BlockSpec((R_KNN, KNN_K), lambda i: (i, 0)),
        out_shape=jax.ShapeDtypeStruct((N_PAD, KNN_K), jnp.int32),
    )(pos16, post, sqcol, sqrow, batcol, batrow)


def _sc_gather(table, idx, cp):
    b_total = idx.shape[0]
    n_workers = 32
    bpw = b_total // n_workers
    chunk = 240
    nch = bpw // chunk

    @functools.partial(
        pl.kernel,
        out_type=jax.ShapeDtypeStruct((b_total, cp), jnp.float32),
        mesh=plsc.VectorSubcoreMesh(core_axis_name="c", subcore_axis_name="s"),
        scratch_types=[
            pltpu.VMEM((chunk,), jnp.int32),
            pltpu.VMEM((chunk, cp), jnp.float32),
            pltpu.SemaphoreType.DMA,
        ],
        compiler_params=pltpu.CompilerParams(use_tc_tiling_on_sc=False),
    )
    def gk(table_hbm, idx_hbm, out_hbm, idx_v, rows_v, sem):
        wid = lax.axis_index("s") * 2 + lax.axis_index("c")
        base = wid * bpw

        def body(c, carry):
            off = base + c * chunk
            pltpu.sync_copy(idx_hbm.at[pl.ds(off, chunk)], idx_v)
            pltpu.async_copy(table_hbm.at[idx_v], rows_v, sem).wait()
            pltpu.sync_copy(rows_v, out_hbm.at[pl.ds(off, chunk)])
            return carry

        lax.fori_loop(0, nch, body, 0)

    return gk(table, idx)


def _make_xconv_body(c_in, c_mid, dm, cp_in):
    def body(pos_ref, posg_ref, xg_ref, w1_ref, w2_ref, mw_ref, ma_ref,
             mb_ref, pcd_ref, p9_ref, dwv_ref, flw_ref, pout_ref, out_ref):
        pos = pos_ref[...]
        rel = posg_ref[...] - jnp.concatenate([pos, pos, pos], axis=1)
        pcd = pcd_ref[...]
        p9 = p9_ref[...]
        w1 = w1_ref[...]
        w2 = w2_ref[...]
        hs = []
        for k in range(3):
            z = _mmb(rel[:, 16 * k:16 * k + 16], w1) + pcd[0:1]
            z = pcd[1:2] * _elu(z) * INV + pcd[2:3]
            z = _mmb(z, w2) + pcd[3:4]
            z = pcd[4:5] * _elu(z) * INV + pcd[5:6]
            hs.append(z)
        t = _mmb(rel, mw_ref[...]) + p9[0:1]
        t = p9[1:2] * _elu(t) * INV + p9[2:3]
        t = _mm(t, ma_ref[...]) + p9[3:4]
        t = p9[4:5] * _elu(t) * INV + p9[5:6]
        t = _mm(t, mb_ref[...]) + p9[6:7]
        t = p9[7:8] * t * INV + p9[8:9]
        xg = xg_ref[...]
        xs = [jnp.concatenate(
            [hs[k], xg[:, cp_in * k:cp_in * k + c_in]], axis=1)
            for k in range(3)]
        xts = []
        for k in range(3):
            acc = xs[0] * t[:, k:k + 1]
            acc = acc + xs[1] * t[:, 3 + k:4 + k]
            acc = acc + xs[2] * t[:, 6 + k:7 + k]
            xts.append(acc)
        dwv = dwv_ref[...]
        pout = pout_ref[...]
        out = None
        for j in range(dm):
            o = (xts[0] * dwv[3 * j:3 * j + 1]
                 + xts[1] * dwv[3 * j + 1:3 * j + 2]
                 + xts[2] * dwv[3 * j + 2:3 * j + 3]
                 + dwv[6 + j:7 + j])
            m = _mmb(o, flw_ref[j * c_mid:(j + 1) * c_mid, :])
            out = m if out is None else out + m
        out = out + pout[0:1]
        out = pout[1:2] * out * INV + pout[2:3]
        out_ref[...] = jnp.maximum(out, 0.0)

    return body


def _prep_xconv(p, outer_g, outer_b, c_in, c_out, c_out_pad):
    cd = c_in // 4
    c_mid = c_in + cd
    dm = -(-c_out // c_mid)
    z32 = jnp.float32
    w1 = jnp.zeros((16, cd), z32).at[:3].set(p['l1_W'].T)
    w2 = p['l2_W'].T
    mw = jnp.zeros((3, 16, 9), z32).at[:, :3, :].set(
        p['m_W'].T.reshape(3, 3, 9)).reshape(48, 9)

    def blockdiag(cw):
        a = jnp.transpose(cw.reshape(3, 3, 3), (0, 2, 1))
        r = jnp.arange(3)
        return jnp.zeros((3, 3, 3, 3), z32).at[r, :, r, :].set(a).reshape(9, 9)

    ma = blockdiag(p['cA_W'])
    mb = blockdiag(p['cB_W'])
    pcd = jnp.zeros((8, cd), z32)
    for i, v in enumerate([p['l1_b'], p['bn1_g'], p['bn1_b'],
                           p['l2_b'], p['bn2_g'], p['bn2_b']]):
        pcd = pcd.at[i].set(v)
    p9 = jnp.zeros((16, 9), z32)
    for i, v in enumerate([p['m_b'], p['mbn_g'], p['mbn_b'],
                           p['cA_b'], p['bnA_g'], p['bnA_b'],
                           p['cB_b'], p['bnB_g'], p['bnB_b']]):
        p9 = p9.at[i].set(v)
    dwr = jnp.transpose(p['dw_W'].reshape(c_mid, dm, 3), (1, 2, 0))
    dwv = jnp.zeros((8, c_mid), z32).at[0:3 * dm].set(dwr.reshape(dm * 3, c_mid))
    dwv = dwv.at[6:6 + dm].set(p['dw_b'].reshape(c_mid, dm).T)
    flr = jnp.transpose(p['fl_W'].T.reshape(c_mid, dm, c_out),
                        (1, 0, 2)).reshape(dm * c_mid, c_out)
    flw = jnp.zeros((dm * c_mid, c_out_pad), z32).at[:, :c_out].set(flr)
    pout = jnp.zeros((8, c_out_pad), z32)
    pout = pout.at[0, :c_out].set(p['fl_b'])
    pout = pout.at[1, :c_out].set(outer_g)
    pout = pout.at[2, :c_out].set(outer_b)
    return (w1, w2, mw, ma, mb, pcd, p9, dwv, flw, pout), cd, c_mid, dm


def _xconv(pos16, posg, xg, prep, c_in, cd, c_mid, dm, cp_in, c_out_pad):
    w1, w2, mw, ma, mb, pcd, p9, dwv, flw, pout = prep

    def full(shape):
        return pl.BlockSpec(shape, lambda i: tuple(0 for _ in shape))

    return pl.pallas_call(
        _make_xconv_body(c_in, c_mid, dm, cp_in),
        grid=(N_PAD // R_XC,),
        in_specs=[
            pl.BlockSpec((R_XC, 16), lambda i: (i, 0)),
            pl.BlockSpec((R_XC, 48), lambda i: (i, 0)),
            pl.BlockSpec((R_XC, 3 * cp_in), lambda i: (i, 0)),
            full((16, cd)), full((cd, cd)), full((48, 9)), full((9, 9)),
            full((9, 9)), full((8, cd)), full((16, 9)), full((8, c_mid)),
            full((dm * c_mid, c_out_pad)), full((8, c_out_pad)),
        ],
        out_specs=pl.BlockSpec((R_XC, c_out_pad), lambda i: (i, 0)),
        out_shape=jax.ShapeDtypeStruct((N_PAD, c_out_pad), jnp.float32),
    )(pos16, posg, xg, w1, w2, mw, ma, mb, pcd, p9, dwv, flw, pout)


def _pool_body(h_ref, b_ref, out_ref):
    @pl.when(pl.program_id(0) == 0)
    def _init():
        out_ref[...] = jnp.zeros_like(out_ref)

    br = b_ref[0]
    gi = lax.broadcasted_iota(jnp.int32, (32, br.shape[1]), 0)
    oh = (gi == br).astype(jnp.float32)
    out_ref[...] += _mm(oh, h_ref[...])


def _pool(h, batr3, c):
    nb = N_PAD // R_XC
    return pl.pallas_call(
        _pool_body,
        grid=(nb,),
        in_specs=[
            pl.BlockSpec((R_XC, c), lambda i: (i, 0)),
            pl.BlockSpec((1, 1, R_XC), lambda i: (i, 0, 0)),
        ],
        out_specs=pl.BlockSpec((32, c), lambda i: (0, 0)),
        out_shape=jax.ShapeDtypeStruct((32, c), jnp.float32),
    )(h, batr3)


def _head_body(g_ref, d_ref, xw_ref, xb_ref, f1a_ref, f1b_ref, b1_ref,
               w2_ref, b2_ref, w3_ref, b3_ref, w4_ref, b4_ref, w5_ref,
               b5_ref, out_ref):
    z = jnp.maximum(_mmb(g_ref[...], xw_ref[...]) + xb_ref[...], 0.0)
    z = jnp.maximum(_mmb(z, f1a_ref[...]) + _mmb(d_ref[...], f1b_ref[...])
                    + b1_ref[...], 0.0)
    z = jnp.maximum(_mmb(z, w2_ref[...]) + b2_ref[...], 0.0)
    z = jnp.maximum(_mmb(z, w3_ref[...]) + b3_ref[...], 0.0)
    z = jnp.maximum(_mmb(z, w4_ref[...]) + b4_ref[...], 0.0)
    out_ref[...] = _mmb(z, w5_ref[...]) + b5_ref[...]


def _head(g, descriptor, params):
    xw = jnp.zeros((g.shape[1], 1024), jnp.float32).at[:300].set(
        params['xfc_W'].T)
    f1 = params['f1_W'].T
    args = (
        g, descriptor, xw, params['xfc_b'][None, :],
        f1[:1024], f1[1024:], params['f1_b'][None, :],
        params['f2_W'].T, params['f2_b'][None, :],
        params['f3_W'].T, params['f3_b'][None, :],
        params['f4_W'].T, params['f4_b'][None, :],
        params['f5_W'].T, params['f5_b'][None, :],
    )
    return pl.pallas_call(
        _head_body,
        out_shape=jax.ShapeDtypeStruct((32, 1), jnp.float32),
    )(*args)


def kernel(x, pos, descriptor, params, batch):
    n = x.shape[0]
    f32 = jnp.float32
    pos16 = jnp.zeros((N_PAD, 16), f32).at[:n, :3].set(pos)
    post = pos16.T
    batch_p = jnp.full((N_PAD,), 99, jnp.int32).at[:n].set(batch)
    batcol = batch_p.reshape(N_PAD, 1)
    batrow = batch_p.reshape(1, N_PAD)
    batr3 = batch_p.reshape(N_PAD // R_XC, 1, R_XC)

    sq = jnp.sum(pos * pos, axis=1)
    sq_p = jnp.zeros((N_PAD,), f32).at[:n].set(sq)
    nbr = _knn(pos16, post, sq_p.reshape(N_PAD, 1), sq_p.reshape(1, N_PAD),
               batcol, batrow)
    idx = jnp.concatenate(
        [nbr[:n].reshape(-1), jnp.zeros((KNN_K * (N_PAD - n),), jnp.int32)])

    posg = _sc_gather(pos16, idx, 16).reshape(N_PAD, 48)

    h = jnp.zeros((N_PAD, LAYERS[0][2]), f32).at[:n, :57].set(x)
    for li, (c_in, c_out, cp_in, c_out_pad) in enumerate(LAYERS):
        lp = params[f'c{li + 1}']
        prep, cd, c_mid, dm = _prep_xconv(
            lp, params[f'bn{li + 1}_g'], params[f'bn{li + 1}_b'],
            c_in, c_out, c_out_pad)
        xg = _sc_gather(h, idx, cp_in).reshape(N_PAD, 3 * cp_in)
        h = _xconv(pos16, posg, xg, prep, c_in, cd, c_mid, dm,
                   cp_in, c_out_pad)

    g = _pool(h, batr3, h.shape[1])
    return _head(g, descriptor, params)
